# Initial kernel scaffold; baseline (speedup 1.0000x reference)
import jax
import jax.numpy as jnp
from jax.experimental import pallas as pl


def kernel(x, edge_index, W1a, b1a, W2a, b2a, W1b, b1b, W2b, b2b):
    raise NotImplementedError("write your pallas kernel here")


# speedup vs baseline: 4.6384x; 4.6384x over previous
import functools

import jax
import jax.numpy as jnp
from jax import lax
from jax.experimental import pallas as pl
from jax.experimental.pallas import tpu as pltpu
from jax.experimental.pallas import tpu_sc as plsc

NC = 2
NS = 16
NW = NC * NS
CHUNK = 128


def _make_sc_agg(n_nodes, d, nchunk, acc_rows):
    mesh = plsc.VectorSubcoreMesh(core_axis_name="c", subcore_axis_name="s")
    rows_per_tile = acc_rows // NS

    @functools.partial(
        pl.kernel,
        mesh=mesh,
        out_type=jax.ShapeDtypeStruct((NC * acc_rows, d), jnp.float32),
        scratch_types=[
            pltpu.VMEM((nchunk, CHUNK), jnp.int32),
            pltpu.VMEM((nchunk, CHUNK), jnp.int32),
            pltpu.VMEM((CHUNK, d), jnp.float32),
            pltpu.VMEM_SHARED((acc_rows, d), jnp.float32),
            pltpu.SemaphoreType.DMA,
        ],
    )
    def agg(table_hbm, src_hbm, dst_hbm, out_hbm, src_v, dst_v, rows_v, acc_sh, sem):
        c = lax.axis_index("c")
        s = lax.axis_index("s")
        wid = s * NC + c

        pltpu.sync_copy(src_hbm.at[wid], src_v)
        pltpu.sync_copy(dst_hbm.at[wid], dst_v)

        zeros = jnp.zeros((16,), jnp.float32)

        def zb(i, carry):
            rows_v[i // (d // 16), pl.ds((i % (d // 16)) * 16, 16)] = zeros
            return carry

        lax.fori_loop(0, CHUNK * (d // 16), zb, 0)
        for blk in range(rows_per_tile // CHUNK):
            pltpu.sync_copy(
                rows_v, acc_sh.at[pl.ds(s * rows_per_tile + blk * CHUNK, CHUNK)]
            )
        plsc.subcore_barrier()

        def body(j, carry):
            pltpu.async_copy(table_hbm.at[src_v.at[j]], rows_v, sem).wait()
            pltpu.sync_copy(rows_v, acc_sh.at[dst_v.at[j]], add=True)
            return carry

        lax.fori_loop(0, nchunk, body, 0)
        plsc.subcore_barrier()

        pltpu.sync_copy(
            acc_sh.at[pl.ds(s * rows_per_tile, rows_per_tile)],
            out_hbm.at[pl.ds(c * acc_rows + s * rows_per_tile, rows_per_tile)],
        )

    return agg


def _mlp_body(x_ref, p0_ref, p1_ref, w1_ref, b1_ref, w2_ref, b2_ref, o_ref,
              *, relu_out):
    z = x_ref[...] + p0_ref[...] + p1_ref[...]
    h = jnp.dot(z, w1_ref[...], preferred_element_type=jnp.float32) + b1_ref[...]
    h = jnp.maximum(h, 0.0)
    o = jnp.dot(h, w2_ref[...], preferred_element_type=jnp.float32) + b2_ref[...]
    o_ref[...] = jnp.maximum(o, 0.0) if relu_out else o


def _make_tc_mlp(n_nodes, d, relu_out, block_rows=1000):
    grid = (n_nodes // block_rows,)
    row_spec = pl.BlockSpec((block_rows, d), lambda i: (i, 0))
    full_spec = pl.BlockSpec((d, d), lambda i: (0, 0))
    bias_spec = pl.BlockSpec((1, d), lambda i: (0, 0))
    return pl.pallas_call(
        functools.partial(_mlp_body, relu_out=relu_out),
        grid=grid,
        in_specs=[row_spec, row_spec, row_spec,
                  full_spec, bias_spec, full_spec, bias_spec],
        out_specs=row_spec,
        out_shape=jax.ShapeDtypeStruct((n_nodes, d), jnp.float32),
    )


def kernel(x, edge_index, W1a, b1a, W2a, b2a, W1b, b1b, W2b, b2b):
    n_nodes, d = x.shape
    n_edges = edge_index.shape[1]

    epw = -(-n_edges // (NW * CHUNK)) * CHUNK
    e_pad = epw * NW
    nchunk = epw // CHUNK
    acc_rows = -(-(n_nodes + 1) // (NS * CHUNK)) * (NS * CHUNK)
    dummy = n_nodes

    pad = e_pad - n_edges
    src = jnp.concatenate(
        [edge_index[0], jnp.zeros((pad,), jnp.int32)]).reshape(NW, nchunk, CHUNK)
    dst = jnp.concatenate(
        [edge_index[1], jnp.full((pad,), dummy, jnp.int32)]).reshape(NW, nchunk, CHUNK)

    sc_agg = _make_sc_agg(n_nodes, d, nchunk, acc_rows)
    mlp1 = _make_tc_mlp(n_nodes, d, relu_out=True)
    mlp2 = _make_tc_mlp(n_nodes, d, relu_out=False)

    b1a_, b2a_ = b1a.reshape(1, d), b2a.reshape(1, d)
    b1b_, b2b_ = b1b.reshape(1, d), b2b.reshape(1, d)

    parts = sc_agg(x, src, dst)
    h1 = mlp1(x, parts[:n_nodes], parts[acc_rows:acc_rows + n_nodes],
              W1a, b1a_, W2a, b2a_)
    parts2 = sc_agg(h1, src, dst)
    out = mlp2(h1, parts2[:n_nodes], parts2[acc_rows:acc_rows + n_nodes],
               W1b, b1b_, W2b, b2b_)
    return out
